# baseline (speedup 1.0000x reference)
import functools

import jax
import jax.numpy as jnp
from jax import lax
from jax.experimental import pallas as pl
from jax.experimental.pallas import tpu as pltpu
from jax.experimental.pallas import tpu_sc as plsc

NUM_CORES = 2
NUM_SUBCORES = 16
LANES = 16
NW = NUM_CORES * NUM_SUBCORES
SUB_BLOCKS = 8


def _make_sc_embed(n_tokens: int, vocab: int, d: int):
    tok_per_w = n_tokens // NW
    n_sb = tok_per_w // SUB_BLOCKS
    assert n_tokens == NW * SUB_BLOCKS * n_sb
    assert n_sb % LANES == 0 and n_sb % 8 == 0
    d2 = 2 * d

    mesh = plsc.VectorSubcoreMesh(core_axis_name="c", subcore_axis_name="s")

    @functools.partial(
        pl.kernel,
        out_type=jax.ShapeDtypeStruct((n_tokens, d2), jnp.float32),
        mesh=mesh,
        scratch_types=[
            pltpu.VMEM((n_sb,), jnp.int32),
            pltpu.VMEM((n_sb, d), jnp.float32),
            pltpu.VMEM((n_sb, d), jnp.float32),
            pltpu.VMEM((n_sb, d2), jnp.float32),
            pltpu.SemaphoreType.DMA,
        ],
        compiler_params=pltpu.CompilerParams(
            use_tc_tiling_on_sc=False, needs_layout_passes=False
        ),
    )
    def sc_embed(f_hbm, a_hbm, idx_hbm, out_hbm, idx_v, rf_v, ra_v, out_v, sem):
        wid = lax.axis_index("s") * NUM_CORES + lax.axis_index("c")
        tok_base = wid * tok_per_w

        ii = lax.iota(jnp.int32, LANES)
        tpat = [(16 * j + ii) // d for j in range(d)]
        cpat = [(16 * j + ii) % d for j in range(d)]
        cpat_hi = [c + d for c in cpat]

        groups = n_sb * d // (d * LANES)

        for sb in range(SUB_BLOCKS):
            base = tok_base + sb * n_sb
            pltpu.sync_copy(idx_hbm.at[pl.ds(base, n_sb)], idx_v)
            cf = pltpu.async_copy(f_hbm.at[idx_v], rf_v, sem)
            ca = pltpu.async_copy(a_hbm.at[idx_v], ra_v, sem)
            cf.wait()
            ca.wait()

            def body(g, carry):
                tbase = g * LANES
                for j in range(d):
                    t_idx = tpat[j] + tbase
                    vf = plsc.load_gather(rf_v, [t_idx, cpat[j]])
                    plsc.store_scatter(out_v, [t_idx, cpat[j]], vf)
                    va = plsc.load_gather(ra_v, [t_idx, cpat[j]])
                    plsc.store_scatter(out_v, [t_idx, cpat_hi[j]], va)
                return carry

            lax.fori_loop(0, groups, body, None)
            pltpu.sync_copy(out_v, out_hbm.at[pl.ds(base, n_sb)])

    return sc_embed


def kernel(token_ids, frequencies, amplitudes):
    b, s = token_ids.shape
    vocab, d = frequencies.shape
    idx = token_ids.reshape(-1).astype(jnp.int32)
    sc_embed = _make_sc_embed(b * s, vocab, d)
    out = sc_embed(frequencies, amplitudes, idx)
    return out.reshape(b, s, 2 * d)


# speedup vs baseline: 20.9728x; 20.9728x over previous
import functools

import jax
import jax.numpy as jnp
from jax import lax
from jax.experimental import pallas as pl
from jax.experimental.pallas import tpu as pltpu
from jax.experimental.pallas import tpu_sc as plsc

NUM_CORES = 2
NUM_SUBCORES = 16
NW = NUM_CORES * NUM_SUBCORES
ROWS_PER_BLK = 8


def _make_sc_embed(seq: int, batch: int, vocab: int, d2: int):
    row_blocks = seq // ROWS_PER_BLK
    workers_per_row = NW // ROWS_PER_BLK
    chunk = batch // workers_per_row
    assert seq == row_blocks * ROWS_PER_BLK and batch == workers_per_row * chunk

    mesh = plsc.VectorSubcoreMesh(core_axis_name="c", subcore_axis_name="s")

    @functools.partial(
        pl.kernel,
        out_type=jax.ShapeDtypeStruct((d2, seq, batch), jnp.float32),
        mesh=mesh,
        scratch_types=[
            pltpu.VMEM((chunk,), jnp.int32),
            [pltpu.VMEM((chunk,), jnp.float32) for _ in range(d2)],
            pltpu.SemaphoreType.DMA,
        ],
    )
    def sc_embed(*refs):
        tables = refs[:d2]
        tok_hbm = refs[d2]
        out_hbm = refs[d2 + 1]
        tok_v = refs[d2 + 2]
        dests = refs[d2 + 3]
        sem = refs[d2 + 4]

        wid = lax.axis_index("s") * NUM_CORES + lax.axis_index("c")
        r_off = wid // workers_per_row
        col0 = (wid % workers_per_row) * chunk

        for si in range(row_blocks):
            s = ROWS_PER_BLK * si + r_off
            pltpu.sync_copy(tok_hbm.at[s, pl.ds(col0, chunk)], tok_v)
            handles = [
                pltpu.async_copy(tables[c].at[tok_v], dests[c], sem)
                for c in range(d2)
            ]
            for h in handles:
                h.wait()
            for c in range(d2):
                pltpu.sync_copy(dests[c], out_hbm.at[c, s, pl.ds(col0, chunk)])

    return sc_embed


def kernel(token_ids, frequencies, amplitudes):
    b, s = token_ids.shape
    vocab, d = frequencies.shape
    tok_t = token_ids.T.astype(jnp.int32)
    cols = [frequencies[:, c] for c in range(d)] + [
        amplitudes[:, c] for c in range(d)
    ]
    sc_embed = _make_sc_embed(s, b, vocab, 2 * d)
    out = sc_embed(*cols, tok_t)
    return out.transpose(2, 1, 0)
